# baseline (speedup 1.0000x reference)
import jax
import jax.numpy as jnp
from jax.experimental import pallas as pl

GRAPH_LAYERS = 3
FLOW_ITERS = 5


def _seg_softmax(vals, seg, n):
    m = jax.ops.segment_max(vals, seg, num_segments=n)
    m = jnp.where(jnp.isfinite(m), m, 0.0)
    ex = jnp.exp(vals - m[seg])
    den = jax.ops.segment_sum(ex, seg, num_segments=n)
    return ex / (den[seg] + 1e-9)


def _cost_kernel(flow_ref, out_ref):
    out_ref[...] = jnp.sum(flow_ref[...] * flow_ref[...]).reshape(1, 1)


def kernel(node_embeddings, demands, edge_index, W_enc, b_enc, W_gat, a_src, a_dst, W_gate, b_gate, W_dec, b_dec):
    n = node_embeddings.shape[0]
    loop = jnp.arange(n, dtype=edge_index.dtype)
    src = jnp.concatenate([edge_index[0], loop])
    dst = jnp.concatenate([edge_index[1], loop])
    x = jnp.concatenate([node_embeddings, demands], axis=1)
    h = jax.nn.relu(x @ W_enc + b_enc)
    num_heads = W_gat.shape[0]
    for _ in range(GRAPH_LAYERS):
        outs = []
        for hd in range(num_heads):
            hf = h @ W_gat[hd]
            e = jax.nn.leaky_relu((hf @ a_src[hd])[src] + (hf @ a_dst[hd])[dst], 0.2)
            w = _seg_softmax(e, dst, n)
            outs.append(jax.ops.segment_sum(hf[src] * w[:, None], dst, num_segments=n))
        nxt = jax.nn.relu(jnp.mean(jnp.stack(outs, axis=0), axis=0))
        z = jax.nn.sigmoid(jnp.concatenate([nxt, h], axis=1) @ W_gate + b_gate)
        h = z * nxt + (1.0 - z) * h
    pred = h @ W_dec + b_dec
    row = edge_index[0]
    col = edge_index[1]
    ev = pred[row, 0]
    w_flow = _seg_softmax(ev, row, n)
    flow = w_flow
    for _ in range(FLOW_ITERS):
        inflow = jax.ops.segment_sum(flow, col, num_segments=n)
        adjusted = jax.nn.relu(inflow - demands[:, 0])
        flow = w_flow * adjusted[row]
    flow_cost = pl.pallas_call(
        _cost_kernel,
        out_shape=jax.ShapeDtypeStruct((1, 1), jnp.float32),
    )(flow.reshape(1, -1))[0, 0]
    return flow_cost, flow


# speedup vs baseline: 1.1572x; 1.1572x over previous
import dataclasses
import functools

import jax
import jax.numpy as jnp
from jax import lax
from jax.experimental import pallas as pl
from jax.experimental.pallas import tpu as pltpu
from jax.experimental.pallas import tpu_sc as plsc

N = 10000
NP = 10240
STRIPE = NP // 16
E = 320000
GRAPH_LAYERS = 3
FLOW_ITERS = 5
NC, NS = 2, 16
B = 128
K2 = 79
T2 = K2 * B
EP2 = 32 * T2

_vmesh = plsc.VectorSubcoreMesh(core_axis_name="c", subcore_axis_name="s")
_sc_params = pltpu.CompilerParams()
if "needs_layout_passes" in pltpu.CompilerParams.__dataclass_fields__:
    _sc_params = dataclasses.replace(_sc_params, needs_layout_passes=False)


def _flow_body(rowp, colp, predt, demt, zerot, flow_out,
               rowv, colv, predv, demv, denv, accv, adjv, wv, stage, mbuf, fbuf,
               shparts, shadj):
    c = lax.axis_index("c")
    s = lax.axis_index("s")

    @pl.when(c == 0)
    def _():
        pltpu.sync_copy(predt, predv)
        pltpu.sync_copy(demt, demv)
        pltpu.sync_copy(zerot, denv)
        pltpu.sync_copy(zerot, accv)

        for j in range(2):
            pltpu.sync_copy(rowp.at[2 * s + j], rowv)

            @pl.loop(0, K2)
            def _(k):
                for g in range(8):
                    idx = rowv[k, pl.ds(g * 16, 16)]
                    ex = jnp.exp(plsc.load_gather(predv, [idx]))
                    plsc.addupdate_scatter(denv, [idx], ex)

        plsc.subcore_barrier()
        pltpu.sync_copy(denv, shparts.at[s])
        plsc.subcore_barrier()
        for p in range(16):
            pltpu.sync_copy(shparts.at[p].at[pl.ds(s * STRIPE, STRIPE)], stage.at[p])

        @pl.loop(0, STRIPE // 16)
        def _(r):
            v = stage[0, pl.ds(r * 16, 16)]
            for p in range(1, 16):
                v = v + stage[p, pl.ds(r * 16, 16)]
            mbuf[pl.ds(r * 16, 16)] = v

        pltpu.sync_copy(mbuf, shadj.at[pl.ds(s * STRIPE, STRIPE)])
        plsc.subcore_barrier()
        pltpu.sync_copy(shadj, denv)

        for j in range(2):
            pltpu.sync_copy(rowp.at[2 * s + j], rowv)
            pltpu.sync_copy(colp.at[2 * s + j], colv)

            @pl.loop(0, K2)
            def _(k):
                for g in range(8):
                    idx = rowv[k, pl.ds(g * 16, 16)]
                    ex = jnp.exp(plsc.load_gather(predv, [idx]))
                    den16 = plsc.load_gather(denv, [idx])
                    w16 = ex / (den16 + 1e-9)
                    wv[pl.ds(j * T2 + k * B + g * 16, 16)] = w16
                    cidx = colv[k, pl.ds(g * 16, 16)]
                    plsc.addupdate_scatter(accv, [cidx], w16)

        for t in range(FLOW_ITERS):
            plsc.subcore_barrier()
            pltpu.sync_copy(accv, shparts.at[s])
            plsc.subcore_barrier()
            for p in range(16):
                pltpu.sync_copy(shparts.at[p].at[pl.ds(s * STRIPE, STRIPE)],
                                stage.at[p])

            @pl.loop(0, STRIPE // 16)
            def _(r):
                v = stage[0, pl.ds(r * 16, 16)]
                for p in range(1, 16):
                    v = v + stage[p, pl.ds(r * 16, 16)]
                d16 = demv[pl.ds(s * STRIPE + r * 16, 16)]
                mbuf[pl.ds(r * 16, 16)] = jnp.maximum(v - d16, 0.0)

            pltpu.sync_copy(mbuf, shadj.at[pl.ds(s * STRIPE, STRIPE)])
            plsc.subcore_barrier()
            pltpu.sync_copy(shadj, adjv)

            if t < FLOW_ITERS - 1:
                pltpu.sync_copy(zerot, accv)
                for j in range(2):
                    pltpu.sync_copy(rowp.at[2 * s + j], rowv)
                    pltpu.sync_copy(colp.at[2 * s + j], colv)

                    @pl.loop(0, K2)
                    def _(k):
                        for g in range(8):
                            idx = rowv[k, pl.ds(g * 16, 16)]
                            a16 = plsc.load_gather(adjv, [idx])
                            w16 = wv[pl.ds(j * T2 + k * B + g * 16, 16)]
                            cidx = colv[k, pl.ds(g * 16, 16)]
                            plsc.addupdate_scatter(accv, [cidx], w16 * a16)
            else:
                for j in range(2):
                    pltpu.sync_copy(rowp.at[2 * s + j], rowv)

                    @pl.loop(0, K2)
                    def _(k):
                        for g in range(8):
                            idx = rowv[k, pl.ds(g * 16, 16)]
                            a16 = plsc.load_gather(adjv, [idx])
                            w16 = wv[pl.ds(j * T2 + k * B + g * 16, 16)]
                            fbuf[pl.ds(g * 16, 16)] = w16 * a16
                        pltpu.sync_copy(
                            fbuf, flow_out.at[pl.ds((2 * s + j) * T2 + k * B, B)])


@jax.jit
def _flow_sc(rowp, colp, predt, demt, zerot):
    kern = pl.kernel(
        _flow_body,
        out_type=jax.ShapeDtypeStruct((EP2,), jnp.float32),
        mesh=_vmesh,
        scratch_types=[
            pltpu.VMEM((K2, B), jnp.int32),
            pltpu.VMEM((K2, B), jnp.int32),
            pltpu.VMEM((NP,), jnp.float32),
            pltpu.VMEM((NP,), jnp.float32),
            pltpu.VMEM((NP,), jnp.float32),
            pltpu.VMEM((NP,), jnp.float32),
            pltpu.VMEM((NP,), jnp.float32),
            pltpu.VMEM((2 * T2,), jnp.float32),
            pltpu.VMEM((16, STRIPE), jnp.float32),
            pltpu.VMEM((STRIPE,), jnp.float32),
            pltpu.VMEM((B,), jnp.float32),
            pltpu.VMEM_SHARED((16, NP), jnp.float32),
            pltpu.VMEM_SHARED((NP,), jnp.float32),
        ],
        compiler_params=_sc_params,
    )
    return kern(rowp, colp, predt, demt, zerot)


def _cost_body(flow_ref, out_ref):
    out_ref[...] = jnp.sum(flow_ref[...] * flow_ref[...]).reshape(1, 1)


@jax.jit
def _flow_cost_tc(flow):
    return pl.pallas_call(
        _cost_body,
        out_shape=jax.ShapeDtypeStruct((1, 1), jnp.float32),
    )(flow.reshape(2500, 128))[0, 0]


def _seg_softmax(vals, seg, n):
    m = jax.ops.segment_max(vals, seg, num_segments=n)
    m = jnp.where(jnp.isfinite(m), m, 0.0)
    ex = jnp.exp(vals - m[seg])
    den = jax.ops.segment_sum(ex, seg, num_segments=n)
    return ex / (den[seg] + 1e-9)


def kernel(node_embeddings, demands, edge_index, W_enc, b_enc, W_gat, a_src,
           a_dst, W_gate, b_gate, W_dec, b_dec):
    n = node_embeddings.shape[0]
    loop = jnp.arange(n, dtype=edge_index.dtype)
    src = jnp.concatenate([edge_index[0], loop])
    dst = jnp.concatenate([edge_index[1], loop])
    x = jnp.concatenate([node_embeddings, demands], axis=1)
    h = jax.nn.relu(x @ W_enc + b_enc)
    num_heads = W_gat.shape[0]
    for _ in range(GRAPH_LAYERS):
        outs = []
        for hd in range(num_heads):
            hf = h @ W_gat[hd]
            e = jax.nn.leaky_relu((hf @ a_src[hd])[src] + (hf @ a_dst[hd])[dst], 0.2)
            w = _seg_softmax(e, dst, n)
            outs.append(jax.ops.segment_sum(hf[src] * w[:, None], dst, num_segments=n))
        nxt = jax.nn.relu(jnp.mean(jnp.stack(outs, axis=0), axis=0))
        z = jax.nn.sigmoid(jnp.concatenate([nxt, h], axis=1) @ W_gate + b_gate)
        h = z * nxt + (1.0 - z) * h
    pred = (h @ W_dec + b_dec)[:, 0]

    pad2 = EP2 - E
    rowp = jnp.concatenate([edge_index[0], jnp.full((pad2,), N, jnp.int32)])
    colp = jnp.concatenate([edge_index[1], jnp.full((pad2,), N, jnp.int32)])
    rowp = rowp.reshape(32, K2, B)
    colp = colp.reshape(32, K2, B)
    predt = jnp.pad(pred, (0, NP - N))
    demt = jnp.pad(demands[:, 0], (0, NP - N))
    zerot = jnp.zeros((NP,), jnp.float32)
    flow = _flow_sc(rowp, colp, predt, demt, zerot)[:E]
    flow_cost = _flow_cost_tc(flow)
    return flow_cost, flow
